# Initial kernel scaffold; baseline (speedup 1.0000x reference)
import jax
import jax.numpy as jnp
from jax.experimental import pallas as pl


def kernel(dense_inputs, sparse_inputs, seq_inputs, item_inputs, W_beh, W_sparse, W_att1, b_att1, W_att2, b_att2, W_att3, b_att3, gamma, beta, W_f1, b_f1, a1, W_f2, b_f2, a2, W_final, b_final):
    raise NotImplementedError("write your pallas kernel here")


# speedup vs baseline: 1.4967x; 1.4967x over previous
import functools

import jax
import jax.numpy as jnp
from jax import lax
from jax.experimental import pallas as pl
from jax.experimental.pallas import tpu as pltpu
from jax.experimental.pallas import tpu_sc as plsc

B = 4096; L = 200; BEH = 2; D = 16; VB = 1000000; VS = 100000; NS = 26; ND = 13
DSEQ = BEH * D
ATT1 = 80; ATT2 = 40
F1 = 256; F2 = 128
DIN_IN = DSEQ + DSEQ + ND + NS * D

NW = 32
N_SEQ = B * L * BEH
N_SP = B * NS
N_IT = B * BEH
SEQ_PW = N_SEQ // NW
SP_PW = N_SP // NW
IT_PW = N_IT // NW
SEQ_CHUNK = 2048
SEQ_CHUNKS = SEQ_PW // SEQ_CHUNK


def _sc_gather_body(tbeh, tsp, idx_seq, idx_sp, idx_it,
                    out_seq, out_sp, out_it,
                    idxs_v, rows_v, idxp_v, rowp_v, idxi_v, rowi_v, sem):
    wid = lax.axis_index("s") * 2 + lax.axis_index("c")

    ibase = wid * IT_PW
    pltpu.sync_copy(idx_it.at[pl.ds(ibase, IT_PW)], idxi_v)
    pltpu.async_copy(tbeh.at[idxi_v], rowi_v, sem).wait()
    pltpu.sync_copy(rowi_v, out_it.at[pl.ds(ibase, IT_PW)])

    pbase = wid * SP_PW
    pltpu.sync_copy(idx_sp.at[pl.ds(pbase, SP_PW)], idxp_v)
    pltpu.async_copy(tsp.at[idxp_v], rowp_v, sem).wait()
    pltpu.sync_copy(rowp_v, out_sp.at[pl.ds(pbase, SP_PW)])

    sbase = wid * SEQ_PW

    def chunk(i, carry):
        off = sbase + i * SEQ_CHUNK
        pltpu.sync_copy(idx_seq.at[pl.ds(off, SEQ_CHUNK)], idxs_v)
        pltpu.async_copy(tbeh.at[idxs_v], rows_v, sem).wait()
        pltpu.sync_copy(rows_v, out_seq.at[pl.ds(off, SEQ_CHUNK)])
        return carry

    lax.fori_loop(0, SEQ_CHUNKS, chunk, 0)


@functools.partial(jax.jit, static_argnames=())
def _sc_gather(tbeh, tsp, idx_seq, idx_sp, idx_it):
    mesh = plsc.VectorSubcoreMesh(core_axis_name="c", subcore_axis_name="s")
    k = pl.kernel(
        _sc_gather_body,
        out_type=(
            jax.ShapeDtypeStruct((N_SEQ, D), jnp.float32),
            jax.ShapeDtypeStruct((N_SP, D), jnp.float32),
            jax.ShapeDtypeStruct((N_IT, D), jnp.float32),
        ),
        mesh=mesh,
        scratch_types=[
            pltpu.VMEM((SEQ_CHUNK,), jnp.int32),
            pltpu.VMEM((SEQ_CHUNK, D), jnp.float32),
            pltpu.VMEM((SP_PW,), jnp.int32),
            pltpu.VMEM((SP_PW, D), jnp.float32),
            pltpu.VMEM((IT_PW,), jnp.int32),
            pltpu.VMEM((IT_PW, D), jnp.float32),
            pltpu.SemaphoreType.DMA,
        ],
        compiler_params=pltpu.CompilerParams(use_tc_tiling_on_sc=False),
    )
    return k(tbeh, tsp, idx_seq, idx_sp, idx_it)


BT = 128
GRID_A = B // BT


def _att_body(ids_ref, seq_ref, item_ref, dense_ref, sp_ref,
              wq_ref, ws_ref, wm_ref, b1_ref, w2_ref, b2_ref, w3_ref, b3_ref,
              user_ref, su_ref, si_ref, sd_ref, ss_ref):
    i = pl.program_id(0)
    seq3 = seq_ref[...]
    item = item_ref[...]
    q3 = jnp.broadcast_to(item[:, None, :], (BT, L, DSEQ))
    seq2 = seq3.reshape(BT * L, DSEQ)
    qs2 = (q3 * seq3).reshape(BT * L, DSEQ)
    qh = jnp.dot(item, wq_ref[...], preferred_element_type=jnp.float32) + b1_ref[...]
    h1 = (jnp.dot(seq2, ws_ref[...], preferred_element_type=jnp.float32)
          + jnp.dot(qs2, wm_ref[...], preferred_element_type=jnp.float32))
    h1 = jnp.maximum(h1.reshape(BT, L, ATT1) + qh[:, None, :], 0.0)
    h2 = jnp.dot(h1.reshape(BT * L, ATT1), w2_ref[...],
                 preferred_element_type=jnp.float32) + b2_ref[...]
    h2 = jnp.maximum(h2, 0.0).reshape(BT, L, ATT2)
    scores = jnp.sum(h2 * w3_ref[...][None, :, :], axis=-1) + b3_ref[0, 0]
    mask = ids_ref[...] != 0
    scores = jnp.where(mask, scores, -4294967295.0)
    m = jnp.max(scores, axis=-1, keepdims=True)
    e = jnp.exp(scores - m)
    w = e / jnp.sum(e, axis=-1, keepdims=True)
    user = jnp.sum(w[:, :, None] * seq3, axis=1)
    user_ref[...] = user

    @pl.when(i == 0)
    def _init():
        su_ref[...] = jnp.zeros_like(su_ref)
        si_ref[...] = jnp.zeros_like(si_ref)
        sd_ref[...] = jnp.zeros_like(sd_ref)
        ss_ref[...] = jnp.zeros_like(ss_ref)

    dense = dense_ref[...]
    sp = sp_ref[...]
    su_ref[0:1, :] += jnp.sum(user, axis=0, keepdims=True)
    su_ref[1:2, :] += jnp.sum(user * user, axis=0, keepdims=True)
    si_ref[0:1, :] += jnp.sum(item, axis=0, keepdims=True)
    si_ref[1:2, :] += jnp.sum(item * item, axis=0, keepdims=True)
    sd_ref[0:1, :] += jnp.sum(dense, axis=0, keepdims=True)
    sd_ref[1:2, :] += jnp.sum(dense * dense, axis=0, keepdims=True)
    ss_ref[0:1, :] += jnp.sum(sp, axis=0, keepdims=True)
    ss_ref[1:2, :] += jnp.sum(sp * sp, axis=0, keepdims=True)


def _attention(ids, seq3, item, dense, sp, wq, ws, wm, b1, w2, b2, w3, b3):
    wspec = lambda shape: pl.BlockSpec(shape, lambda i: (0,) * len(shape))
    return pl.pallas_call(
        _att_body,
        grid=(GRID_A,),
        in_specs=[
            pl.BlockSpec((BT, L), lambda i: (i, 0)),
            pl.BlockSpec((BT, L, DSEQ), lambda i: (i, 0, 0)),
            pl.BlockSpec((BT, DSEQ), lambda i: (i, 0)),
            pl.BlockSpec((BT, ND), lambda i: (i, 0)),
            pl.BlockSpec((BT, NS * D), lambda i: (i, 0)),
            wspec((DSEQ, ATT1)), wspec((DSEQ, ATT1)), wspec((DSEQ, ATT1)),
            wspec((1, ATT1)), wspec((ATT1, ATT2)), wspec((1, ATT2)),
            wspec((1, ATT2)), wspec((1, 1)),
        ],
        out_specs=[
            pl.BlockSpec((BT, DSEQ), lambda i: (i, 0)),
            pl.BlockSpec((2, DSEQ), lambda i: (0, 0)),
            pl.BlockSpec((2, DSEQ), lambda i: (0, 0)),
            pl.BlockSpec((2, ND), lambda i: (0, 0)),
            pl.BlockSpec((2, NS * D), lambda i: (0, 0)),
        ],
        out_shape=[
            jax.ShapeDtypeStruct((B, DSEQ), jnp.float32),
            jax.ShapeDtypeStruct((2, DSEQ), jnp.float32),
            jax.ShapeDtypeStruct((2, DSEQ), jnp.float32),
            jax.ShapeDtypeStruct((2, ND), jnp.float32),
            jax.ShapeDtypeStruct((2, NS * D), jnp.float32),
        ],
    )(ids, seq3, item, dense, sp, wq, ws, wm, b1, w2, b2, w3, b3)


BT2 = 512
GRID_B = B // BT2


def _mlp_body(user_ref, item_ref, dense_ref, sp_ref,
              su_ref, si_ref, sd_ref, ss_ref,
              gu_ref, gi_ref, gd_ref, gs_ref,
              bu_ref, bi_ref, bd_ref, bs_ref,
              w1u_ref, w1i_ref, w1d_ref, w1s_ref, b1_ref, a1_ref,
              w2_ref, b2_ref, a2_ref, wf_ref, bf_ref, out_ref):
    inv_b = 1.0 / B

    def norm(x, s_ref, g_ref, b_ref):
        mu = s_ref[0:1, :] * inv_b
        var = s_ref[1:2, :] * inv_b - mu * mu
        return g_ref[...] * (x - mu) * lax.rsqrt(var + 1e-3) + b_ref[...]

    xu = norm(user_ref[...], su_ref, gu_ref, bu_ref)
    xi = norm(item_ref[...], si_ref, gi_ref, bi_ref)
    xd = norm(dense_ref[...], sd_ref, gd_ref, bd_ref)
    xs = norm(sp_ref[...], ss_ref, gs_ref, bs_ref)
    h = (jnp.dot(xu, w1u_ref[...], preferred_element_type=jnp.float32)
         + jnp.dot(xi, w1i_ref[...], preferred_element_type=jnp.float32)
         + jnp.dot(xd, w1d_ref[...], preferred_element_type=jnp.float32)
         + jnp.dot(xs, w1s_ref[...], preferred_element_type=jnp.float32)
         + b1_ref[...])
    h = jnp.maximum(h, 0.0) + a1_ref[...] * jnp.minimum(h, 0.0)
    h = jnp.dot(h, w2_ref[...], preferred_element_type=jnp.float32) + b2_ref[...]
    h = jnp.maximum(h, 0.0) + a2_ref[...] * jnp.minimum(h, 0.0)
    o = jnp.dot(h, wf_ref[...], preferred_element_type=jnp.float32) + bf_ref[0, 0]
    out_ref[...] = 1.0 / (1.0 + jnp.exp(-o))


def _mlp(user, item, dense, sp, su, si, sd, ss, gb, w1p, b1, a1, w2, b2, a2, wf, bf):
    gu, gi, gd, gs, bu, bi, bd, bs = gb
    w1u, w1i, w1d, w1s = w1p
    wspec = lambda shape: pl.BlockSpec(shape, lambda i: (0,) * len(shape))
    return pl.pallas_call(
        _mlp_body,
        grid=(GRID_B,),
        in_specs=[
            pl.BlockSpec((BT2, DSEQ), lambda i: (i, 0)),
            pl.BlockSpec((BT2, DSEQ), lambda i: (i, 0)),
            pl.BlockSpec((BT2, ND), lambda i: (i, 0)),
            pl.BlockSpec((BT2, NS * D), lambda i: (i, 0)),
            wspec((2, DSEQ)), wspec((2, DSEQ)), wspec((2, ND)), wspec((2, NS * D)),
            wspec((1, DSEQ)), wspec((1, DSEQ)), wspec((1, ND)), wspec((1, NS * D)),
            wspec((1, DSEQ)), wspec((1, DSEQ)), wspec((1, ND)), wspec((1, NS * D)),
            wspec((DSEQ, F1)), wspec((DSEQ, F1)), wspec((ND, F1)), wspec((NS * D, F1)),
            wspec((1, F1)), wspec((1, F1)),
            wspec((F1, F2)), wspec((1, F2)), wspec((1, F2)),
            wspec((F2, 1)), wspec((1, 1)),
        ],
        out_specs=pl.BlockSpec((BT2, 1), lambda i: (i, 0)),
        out_shape=jax.ShapeDtypeStruct((B, 1), jnp.float32),
    )(user, item, dense, sp, su, si, sd, ss, gu, gi, gd, gs, bu, bi, bd, bs,
      w1u, w1i, w1d, w1s, b1, a1, w2, b2, a2, wf, bf)


def kernel(dense_inputs, sparse_inputs, seq_inputs, item_inputs, W_beh, W_sparse,
           W_att1, b_att1, W_att2, b_att2, W_att3, b_att3, gamma, beta,
           W_f1, b_f1, a1, W_f2, b_f2, a2, W_final, b_final):
    tbeh = W_beh.reshape(BEH * VB, D)
    tsp = W_sparse.reshape(NS * VS, D)
    beh_off = (jnp.arange(BEH, dtype=jnp.int32) * VB)
    idx_seq = (seq_inputs.astype(jnp.int32) + beh_off[None, None, :]).reshape(-1)
    idx_it = (item_inputs.astype(jnp.int32) + beh_off[None, :]).reshape(-1)
    sp_off = (jnp.arange(NS, dtype=jnp.int32) * VS)
    idx_sp = (sparse_inputs.astype(jnp.int32) + sp_off[None, :]).reshape(-1)

    rows_seq, rows_sp, rows_it = _sc_gather(tbeh, tsp, idx_seq, idx_sp, idx_it)
    seq3 = rows_seq.reshape(B, L, DSEQ)
    item = rows_it.reshape(B, DSEQ)
    sp = rows_sp.reshape(B, NS * D)

    W1 = W_att1
    wq = W1[0:DSEQ] + W1[2 * DSEQ:3 * DSEQ]
    ws = W1[DSEQ:2 * DSEQ] - W1[2 * DSEQ:3 * DSEQ]
    wm = W1[3 * DSEQ:4 * DSEQ]
    ids = seq_inputs[:, :, 0].astype(jnp.int32)

    user, su, si, sd, ss = _attention(
        ids, seq3, item, dense_inputs, sp,
        wq, ws, wm, b_att1.reshape(1, ATT1), W_att2, b_att2.reshape(1, ATT2),
        W_att3.reshape(1, ATT2), b_att3.reshape(1, 1))

    o0, o1, o2, o3 = 0, DSEQ, 2 * DSEQ, 2 * DSEQ + ND
    gb = (gamma[o0:o1].reshape(1, -1), gamma[o1:o2].reshape(1, -1),
          gamma[o2:o3].reshape(1, -1), gamma[o3:].reshape(1, -1),
          beta[o0:o1].reshape(1, -1), beta[o1:o2].reshape(1, -1),
          beta[o2:o3].reshape(1, -1), beta[o3:].reshape(1, -1))
    w1p = (W_f1[o0:o1], W_f1[o1:o2], W_f1[o2:o3], W_f1[o3:])

    return _mlp(user, item, dense_inputs, sp, su, si, sd, ss, gb, w1p,
                b_f1.reshape(1, F1), a1.reshape(1, F1), W_f2,
                b_f2.reshape(1, F2), a2.reshape(1, F2), W_final,
                b_final.reshape(1, 1))


# speedup vs baseline: 1.9646x; 1.3126x over previous
import functools

import jax
import jax.numpy as jnp
from jax import lax
from jax.experimental import pallas as pl
from jax.experimental.pallas import tpu as pltpu
from jax.experimental.pallas import tpu_sc as plsc

B = 4096; L = 200; BEH = 2; D = 16; VB = 1000000; VS = 100000; NS = 26; ND = 13
DSEQ = BEH * D
ATT1 = 80; ATT2 = 40
F1 = 256; F2 = 128
DIN_IN = DSEQ + DSEQ + ND + NS * D

NW = 32
N_SEQ = B * L * BEH
N_SP = B * NS
N_IT = B * BEH
SEQ_PW = N_SEQ // NW
SP_PW = N_SP // NW
IT_PW = N_IT // NW
SEQ_CHUNK = 2048
SEQ_CHUNKS = SEQ_PW // SEQ_CHUNK


def _sc_gather_body(tbeh, tsp, idx_seq, oidx_seq, idx_sp, oidx_sp, idx_it, oidx_it,
                    out_seq, out_sp, out_it,
                    idxs_v, oidxs_v, rows_v, idxp_v, oidxp_v, rowp_v,
                    idxi_v, oidxi_v, rowi_v, sem):
    wid = lax.axis_index("s") * 2 + lax.axis_index("c")

    ibase = wid * IT_PW
    pltpu.sync_copy(idx_it.at[pl.ds(ibase, IT_PW)], idxi_v)
    pltpu.sync_copy(oidx_it.at[pl.ds(ibase, IT_PW)], oidxi_v)
    pltpu.async_copy(tbeh.at[idxi_v], rowi_v, sem).wait()
    pltpu.async_copy(rowi_v, out_it.at[oidxi_v], sem).wait()

    pbase = wid * SP_PW
    pltpu.sync_copy(idx_sp.at[pl.ds(pbase, SP_PW)], idxp_v)
    pltpu.sync_copy(oidx_sp.at[pl.ds(pbase, SP_PW)], oidxp_v)
    pltpu.async_copy(tsp.at[idxp_v], rowp_v, sem).wait()
    pltpu.async_copy(rowp_v, out_sp.at[oidxp_v], sem).wait()

    sbase = wid * SEQ_PW

    def chunk(i, carry):
        off = sbase + i * SEQ_CHUNK
        pltpu.sync_copy(idx_seq.at[pl.ds(off, SEQ_CHUNK)], idxs_v)
        pltpu.sync_copy(oidx_seq.at[pl.ds(off, SEQ_CHUNK)], oidxs_v)
        pltpu.async_copy(tbeh.at[idxs_v], rows_v, sem).wait()
        pltpu.async_copy(rows_v, out_seq.at[oidxs_v], sem).wait()
        return carry

    lax.fori_loop(0, SEQ_CHUNKS, chunk, 0)


@functools.partial(jax.jit, static_argnames=())
def _sc_gather(tbeh, tsp, idx_seq, oidx_seq, idx_sp, oidx_sp, idx_it, oidx_it):
    mesh = plsc.VectorSubcoreMesh(core_axis_name="c", subcore_axis_name="s")
    k = pl.kernel(
        _sc_gather_body,
        out_type=(
            jax.ShapeDtypeStruct((N_SEQ, D), jnp.float32),
            jax.ShapeDtypeStruct((N_SP, D), jnp.float32),
            jax.ShapeDtypeStruct((N_IT, D), jnp.float32),
        ),
        mesh=mesh,
        scratch_types=[
            pltpu.VMEM((SEQ_CHUNK,), jnp.int32),
            pltpu.VMEM((SEQ_CHUNK,), jnp.int32),
            pltpu.VMEM((SEQ_CHUNK, D), jnp.float32),
            pltpu.VMEM((SP_PW,), jnp.int32),
            pltpu.VMEM((SP_PW,), jnp.int32),
            pltpu.VMEM((SP_PW, D), jnp.float32),
            pltpu.VMEM((IT_PW,), jnp.int32),
            pltpu.VMEM((IT_PW,), jnp.int32),
            pltpu.VMEM((IT_PW, D), jnp.float32),
            pltpu.SemaphoreType.DMA,
        ],
        compiler_params=pltpu.CompilerParams(use_tc_tiling_on_sc=False),
    )
    return k(tbeh, tsp, idx_seq, oidx_seq, idx_sp, oidx_sp, idx_it, oidx_it)


BT = 128
GRID_A = B // BT


def _att_body(ids_ref, seq_ref, item_ref, dense_ref, sp_ref,
              wq_ref, ws_ref, wm_ref, b1_ref, w2_ref, b2_ref, w3_ref, b3_ref,
              user_ref, su_ref, si_ref, sd_ref, ss_ref):
    i = pl.program_id(0)
    seq3 = seq_ref[...]
    item = item_ref[...]
    q3 = jnp.broadcast_to(item[:, None, :], (BT, L, DSEQ))
    seq2 = seq3.reshape(BT * L, DSEQ)
    qs2 = (q3 * seq3).reshape(BT * L, DSEQ)
    qh = jnp.dot(item, wq_ref[...], preferred_element_type=jnp.float32) + b1_ref[...]
    h1 = (jnp.dot(seq2, ws_ref[...], preferred_element_type=jnp.float32)
          + jnp.dot(qs2, wm_ref[...], preferred_element_type=jnp.float32))
    h1 = jnp.maximum(h1.reshape(BT, L, ATT1) + qh[:, None, :], 0.0)
    h2 = jnp.dot(h1.reshape(BT * L, ATT1), w2_ref[...],
                 preferred_element_type=jnp.float32) + b2_ref[...]
    h2 = jnp.maximum(h2, 0.0).reshape(BT, L, ATT2)
    scores = jnp.sum(h2 * w3_ref[...][None, :, :], axis=-1) + b3_ref[0, 0]
    mask = jnp.transpose(ids_ref[...]) != 0
    scores = jnp.where(mask, scores, -4294967295.0)
    m = jnp.max(scores, axis=-1, keepdims=True)
    e = jnp.exp(scores - m)
    w = e / jnp.sum(e, axis=-1, keepdims=True)
    user = jnp.sum(w[:, :, None] * seq3, axis=1)
    user_ref[...] = user

    @pl.when(i == 0)
    def _init():
        su_ref[...] = jnp.zeros_like(su_ref)
        si_ref[...] = jnp.zeros_like(si_ref)
        sd_ref[...] = jnp.zeros_like(sd_ref)
        ss_ref[...] = jnp.zeros_like(ss_ref)

    dense = dense_ref[...]
    sp = sp_ref[...]
    su_ref[0:1, :] += jnp.sum(user, axis=0, keepdims=True)
    su_ref[1:2, :] += jnp.sum(user * user, axis=0, keepdims=True)
    si_ref[0:1, :] += jnp.sum(item, axis=0, keepdims=True)
    si_ref[1:2, :] += jnp.sum(item * item, axis=0, keepdims=True)
    sd_ref[0:1, :] += jnp.sum(dense, axis=0, keepdims=True)
    sd_ref[1:2, :] += jnp.sum(dense * dense, axis=0, keepdims=True)
    ss_ref[0:1, :] += jnp.sum(sp, axis=0, keepdims=True)
    ss_ref[1:2, :] += jnp.sum(sp * sp, axis=0, keepdims=True)


def _attention(ids, seq3, item, dense, sp, wq, ws, wm, b1, w2, b2, w3, b3):
    wspec = lambda shape: pl.BlockSpec(shape, lambda i: (0,) * len(shape))
    return pl.pallas_call(
        _att_body,
        grid=(GRID_A,),
        in_specs=[
            pl.BlockSpec((L, BT), lambda i: (0, i)),
            pl.BlockSpec((BT, L, DSEQ), lambda i: (i, 0, 0)),
            pl.BlockSpec((BT, DSEQ), lambda i: (i, 0)),
            pl.BlockSpec((BT, ND), lambda i: (i, 0)),
            pl.BlockSpec((BT, NS * D), lambda i: (i, 0)),
            wspec((DSEQ, ATT1)), wspec((DSEQ, ATT1)), wspec((DSEQ, ATT1)),
            wspec((1, ATT1)), wspec((ATT1, ATT2)), wspec((1, ATT2)),
            wspec((1, ATT2)), wspec((1, 1)),
        ],
        out_specs=[
            pl.BlockSpec((BT, DSEQ), lambda i: (i, 0)),
            pl.BlockSpec((2, DSEQ), lambda i: (0, 0)),
            pl.BlockSpec((2, DSEQ), lambda i: (0, 0)),
            pl.BlockSpec((2, ND), lambda i: (0, 0)),
            pl.BlockSpec((2, NS * D), lambda i: (0, 0)),
        ],
        out_shape=[
            jax.ShapeDtypeStruct((B, DSEQ), jnp.float32),
            jax.ShapeDtypeStruct((2, DSEQ), jnp.float32),
            jax.ShapeDtypeStruct((2, DSEQ), jnp.float32),
            jax.ShapeDtypeStruct((2, ND), jnp.float32),
            jax.ShapeDtypeStruct((2, NS * D), jnp.float32),
        ],
    )(ids, seq3, item, dense, sp, wq, ws, wm, b1, w2, b2, w3, b3)


BT2 = 512
GRID_B = B // BT2


def _mlp_body(user_ref, item_ref, dense_ref, sp_ref,
              su_ref, si_ref, sd_ref, ss_ref,
              gu_ref, gi_ref, gd_ref, gs_ref,
              bu_ref, bi_ref, bd_ref, bs_ref,
              w1u_ref, w1i_ref, w1d_ref, w1s_ref, b1_ref, a1_ref,
              w2_ref, b2_ref, a2_ref, wf_ref, bf_ref, out_ref):
    inv_b = 1.0 / B

    def norm(x, s_ref, g_ref, b_ref):
        mu = s_ref[0:1, :] * inv_b
        var = s_ref[1:2, :] * inv_b - mu * mu
        return g_ref[...] * (x - mu) * lax.rsqrt(var + 1e-3) + b_ref[...]

    xu = norm(user_ref[...], su_ref, gu_ref, bu_ref)
    xi = norm(item_ref[...], si_ref, gi_ref, bi_ref)
    xd = norm(dense_ref[...], sd_ref, gd_ref, bd_ref)
    xs = norm(sp_ref[...], ss_ref, gs_ref, bs_ref)
    h = (jnp.dot(xu, w1u_ref[...], preferred_element_type=jnp.float32)
         + jnp.dot(xi, w1i_ref[...], preferred_element_type=jnp.float32)
         + jnp.dot(xd, w1d_ref[...], preferred_element_type=jnp.float32)
         + jnp.dot(xs, w1s_ref[...], preferred_element_type=jnp.float32)
         + b1_ref[...])
    h = jnp.maximum(h, 0.0) + a1_ref[...] * jnp.minimum(h, 0.0)
    h = jnp.dot(h, w2_ref[...], preferred_element_type=jnp.float32) + b2_ref[...]
    h = jnp.maximum(h, 0.0) + a2_ref[...] * jnp.minimum(h, 0.0)
    o = jnp.dot(h, wf_ref[...], preferred_element_type=jnp.float32) + bf_ref[0, 0]
    out_ref[...] = 1.0 / (1.0 + jnp.exp(-o))


def _mlp(user, item, dense, sp, su, si, sd, ss, gb, w1p, b1, a1, w2, b2, a2, wf, bf):
    gu, gi, gd, gs, bu, bi, bd, bs = gb
    w1u, w1i, w1d, w1s = w1p
    wspec = lambda shape: pl.BlockSpec(shape, lambda i: (0,) * len(shape))
    return pl.pallas_call(
        _mlp_body,
        grid=(GRID_B,),
        in_specs=[
            pl.BlockSpec((BT2, DSEQ), lambda i: (i, 0)),
            pl.BlockSpec((BT2, DSEQ), lambda i: (i, 0)),
            pl.BlockSpec((BT2, ND), lambda i: (i, 0)),
            pl.BlockSpec((BT2, NS * D), lambda i: (i, 0)),
            wspec((2, DSEQ)), wspec((2, DSEQ)), wspec((2, ND)), wspec((2, NS * D)),
            wspec((1, DSEQ)), wspec((1, DSEQ)), wspec((1, ND)), wspec((1, NS * D)),
            wspec((1, DSEQ)), wspec((1, DSEQ)), wspec((1, ND)), wspec((1, NS * D)),
            wspec((DSEQ, F1)), wspec((DSEQ, F1)), wspec((ND, F1)), wspec((NS * D, F1)),
            wspec((1, F1)), wspec((1, F1)),
            wspec((F1, F2)), wspec((1, F2)), wspec((1, F2)),
            wspec((F2, 1)), wspec((1, 1)),
        ],
        out_specs=pl.BlockSpec((BT2, 1), lambda i: (i, 0)),
        out_shape=jax.ShapeDtypeStruct((B, 1), jnp.float32),
    )(user, item, dense, sp, su, si, sd, ss, gu, gi, gd, gs, bu, bi, bd, bs,
      w1u, w1i, w1d, w1s, b1, a1, w2, b2, a2, wf, bf)


def kernel(dense_inputs, sparse_inputs, seq_inputs, item_inputs, W_beh, W_sparse,
           W_att1, b_att1, W_att2, b_att2, W_att3, b_att3, gamma, beta,
           W_f1, b_f1, a1, W_f2, b_f2, a2, W_final, b_final):
    tbeh = W_beh.reshape(BEH * VB, D)
    tsp = W_sparse.reshape(NS * VS, D)
    beh_off = (jnp.arange(BEH, dtype=jnp.int32) * VB)
    bcol = jnp.arange(B, dtype=jnp.int32)[None, :]
    NBT = B // 128
    seq4 = seq_inputs.astype(jnp.int32).reshape(NBT, 128, L, BEH).transpose(2, 0, 3, 1)
    idx_seq = (seq4 + beh_off[None, None, :, None]).reshape(-1)
    l_ = jnp.arange(L, dtype=jnp.int32)[:, None, None, None]
    bt_ = jnp.arange(NBT, dtype=jnp.int32)[None, :, None, None]
    beh_ = jnp.arange(BEH, dtype=jnp.int32)[None, None, :, None]
    bl_ = jnp.arange(128, dtype=jnp.int32)[None, None, None, :]
    oidx_seq = ((bt_ * 128 + bl_) * (L * BEH) + l_ * BEH + beh_).reshape(-1)
    idx_it = (item_inputs.astype(jnp.int32).T + beh_off[:, None]).reshape(-1)
    oidx_it = (bcol * BEH + jnp.arange(BEH, dtype=jnp.int32)[:, None]).reshape(-1)
    sp_off = (jnp.arange(NS, dtype=jnp.int32) * VS)
    idx_sp = (sparse_inputs.astype(jnp.int32).T + sp_off[:, None]).reshape(-1)
    oidx_sp = (bcol * NS + jnp.arange(NS, dtype=jnp.int32)[:, None]).reshape(-1)

    rows_seq, rows_sp, rows_it = _sc_gather(
        tbeh, tsp, idx_seq, oidx_seq, idx_sp, oidx_sp, idx_it, oidx_it)
    seq3 = rows_seq.reshape(B, L, DSEQ)
    item = rows_it.reshape(B, DSEQ)
    sp = rows_sp.reshape(B, NS * D)

    W1 = W_att1
    wq = W1[0:DSEQ] + W1[2 * DSEQ:3 * DSEQ]
    ws = W1[DSEQ:2 * DSEQ] - W1[2 * DSEQ:3 * DSEQ]
    wm = W1[3 * DSEQ:4 * DSEQ]
    ids = seq_inputs[:, :, 0].astype(jnp.int32).T

    user, su, si, sd, ss = _attention(
        ids, seq3, item, dense_inputs, sp,
        wq, ws, wm, b_att1.reshape(1, ATT1), W_att2, b_att2.reshape(1, ATT2),
        W_att3.reshape(1, ATT2), b_att3.reshape(1, 1))

    o0, o1, o2, o3 = 0, DSEQ, 2 * DSEQ, 2 * DSEQ + ND
    gb = (gamma[o0:o1].reshape(1, -1), gamma[o1:o2].reshape(1, -1),
          gamma[o2:o3].reshape(1, -1), gamma[o3:].reshape(1, -1),
          beta[o0:o1].reshape(1, -1), beta[o1:o2].reshape(1, -1),
          beta[o2:o3].reshape(1, -1), beta[o3:].reshape(1, -1))
    w1p = (W_f1[o0:o1], W_f1[o1:o2], W_f1[o2:o3], W_f1[o3:])

    return _mlp(user, item, dense_inputs, sp, su, si, sd, ss, gb, w1p,
                b_f1.reshape(1, F1), a1.reshape(1, F1), W_f2,
                b_f2.reshape(1, F2), a2.reshape(1, F2), W_final,
                b_final.reshape(1, 1))


# speedup vs baseline: 2.4290x; 1.2364x over previous
import functools

import jax
import jax.numpy as jnp
from jax import lax
from jax.experimental import pallas as pl
from jax.experimental.pallas import tpu as pltpu
from jax.experimental.pallas import tpu_sc as plsc

B = 4096; L = 200; BEH = 2; D = 16; VB = 1000000; VS = 100000; NS = 26; ND = 13
DSEQ = BEH * D
ATT1 = 80; ATT2 = 40
F1 = 256; F2 = 128
DIN_IN = DSEQ + DSEQ + ND + NS * D

NW = 32
N_SEQ = B * L * BEH
N_SP = B * NS
N_IT = B * BEH
SEQ_PW = N_SEQ // NW
SP_PW = N_SP // NW
IT_PW = N_IT // NW
SEQ_CHUNK = 2048
SEQ_CHUNKS = SEQ_PW // SEQ_CHUNK


def _sc_gather_body(tbeh, tsp, idx_seq, oidx_seq, idx_sp, oidx_sp, idx_it, oidx_it,
                    out_seq, out_sp, out_it,
                    idxs_v, oidxs_v, rows_v, idxp_v, oidxp_v, rowp_v,
                    idxi_v, oidxi_v, rowi_v, sem):
    wid = lax.axis_index("s") * 2 + lax.axis_index("c")

    ibase = wid * IT_PW
    pltpu.sync_copy(idx_it.at[pl.ds(ibase, IT_PW)], idxi_v)
    pltpu.sync_copy(oidx_it.at[pl.ds(ibase, IT_PW)], oidxi_v)
    pltpu.async_copy(tbeh.at[idxi_v], rowi_v, sem).wait()
    pltpu.async_copy(rowi_v, out_it.at[oidxi_v], sem).wait()

    pbase = wid * SP_PW
    pltpu.sync_copy(idx_sp.at[pl.ds(pbase, SP_PW)], idxp_v)
    pltpu.sync_copy(oidx_sp.at[pl.ds(pbase, SP_PW)], oidxp_v)
    pltpu.async_copy(tsp.at[idxp_v], rowp_v, sem).wait()
    pltpu.async_copy(rowp_v, out_sp.at[oidxp_v], sem).wait()

    sbase = wid * SEQ_PW

    def chunk(i, carry):
        off = sbase + i * SEQ_CHUNK
        pltpu.sync_copy(idx_seq.at[pl.ds(off, SEQ_CHUNK)], idxs_v)
        pltpu.sync_copy(oidx_seq.at[pl.ds(off, SEQ_CHUNK)], oidxs_v)
        pltpu.async_copy(tbeh.at[idxs_v], rows_v, sem).wait()
        pltpu.async_copy(rows_v, out_seq.at[oidxs_v], sem).wait()
        return carry

    lax.fori_loop(0, SEQ_CHUNKS, chunk, 0)


@functools.partial(jax.jit, static_argnames=())
def _sc_gather(tbeh, tsp, idx_seq, oidx_seq, idx_sp, oidx_sp, idx_it, oidx_it):
    mesh = plsc.VectorSubcoreMesh(core_axis_name="c", subcore_axis_name="s")
    k = pl.kernel(
        _sc_gather_body,
        out_type=(
            jax.ShapeDtypeStruct((N_SEQ, D), jnp.float32),
            jax.ShapeDtypeStruct((N_SP, D), jnp.float32),
            jax.ShapeDtypeStruct((N_IT, D), jnp.float32),
        ),
        mesh=mesh,
        scratch_types=[
            pltpu.VMEM((SEQ_CHUNK,), jnp.int32),
            pltpu.VMEM((SEQ_CHUNK,), jnp.int32),
            pltpu.VMEM((SEQ_CHUNK, D), jnp.float32),
            pltpu.VMEM((SP_PW,), jnp.int32),
            pltpu.VMEM((SP_PW,), jnp.int32),
            pltpu.VMEM((SP_PW, D), jnp.float32),
            pltpu.VMEM((IT_PW,), jnp.int32),
            pltpu.VMEM((IT_PW,), jnp.int32),
            pltpu.VMEM((IT_PW, D), jnp.float32),
            pltpu.SemaphoreType.DMA,
        ],
        compiler_params=pltpu.CompilerParams(use_tc_tiling_on_sc=False),
    )
    return k(tbeh, tsp, idx_seq, oidx_seq, idx_sp, oidx_sp, idx_it, oidx_it)


VC = 8192
VPB = ((VB + VC - 1) // VC) * VC
VPS = ((VS + VC - 1) // VC) * VC


def _repack_body(in_ref, out_ref):
    x = in_ref[...]
    xt3 = jnp.transpose(x[0]).reshape(VC // 8, 8, D)
    for j in range(8):
        out_ref[:, j * D:(j + 1) * D] = xt3[:, j, :]


def _repack(view, G, V, VP):
    nc = VP // VC
    return pl.pallas_call(
        _repack_body,
        grid=(G, nc),
        in_specs=[pl.BlockSpec((1, D, VC), lambda g, c: (g, 0, c))],
        out_specs=pl.BlockSpec((VC // 8, 8 * D), lambda g, c: (g * nc + c, 0)),
        out_shape=jax.ShapeDtypeStruct((G * VP // 8, 8 * D), jnp.float32),
    )(view)


BT = 128
GRID_A = B // BT


def _att_body(ids_ref, seq_ref, item_ref, dense_ref, sp_ref,
              wq_ref, ws_ref, wm_ref, b1_ref, w2_ref, b2_ref, w3_ref, b3_ref,
              user_ref, su_ref, si_ref, sd_ref, ss_ref):
    i = pl.program_id(0)
    seq3 = seq_ref[...]
    item = item_ref[...]
    q3 = jnp.broadcast_to(item[:, None, :], (BT, L, DSEQ))
    seq2 = seq3.reshape(BT * L, DSEQ)
    qs2 = (q3 * seq3).reshape(BT * L, DSEQ)
    qh = jnp.dot(item, wq_ref[...], preferred_element_type=jnp.float32) + b1_ref[...]
    h1 = (jnp.dot(seq2, ws_ref[...], preferred_element_type=jnp.float32)
          + jnp.dot(qs2, wm_ref[...], preferred_element_type=jnp.float32))
    h1 = jnp.maximum(h1.reshape(BT, L, ATT1) + qh[:, None, :], 0.0)
    h2 = jnp.dot(h1.reshape(BT * L, ATT1), w2_ref[...],
                 preferred_element_type=jnp.float32) + b2_ref[...]
    h2 = jnp.maximum(h2, 0.0).reshape(BT, L, ATT2)
    scores = jnp.sum(h2 * w3_ref[...][None, :, :], axis=-1) + b3_ref[0, 0]
    mask = jnp.transpose(ids_ref[...]) != 0
    scores = jnp.where(mask, scores, -4294967295.0)
    m = jnp.max(scores, axis=-1, keepdims=True)
    e = jnp.exp(scores - m)
    w = e / jnp.sum(e, axis=-1, keepdims=True)
    user = jnp.sum(w[:, :, None] * seq3, axis=1)
    user_ref[...] = user

    @pl.when(i == 0)
    def _init():
        su_ref[...] = jnp.zeros_like(su_ref)
        si_ref[...] = jnp.zeros_like(si_ref)
        sd_ref[...] = jnp.zeros_like(sd_ref)
        ss_ref[...] = jnp.zeros_like(ss_ref)

    dense = dense_ref[...]
    sp = sp_ref[...]
    su_ref[0:1, :] += jnp.sum(user, axis=0, keepdims=True)
    su_ref[1:2, :] += jnp.sum(user * user, axis=0, keepdims=True)
    si_ref[0:1, :] += jnp.sum(item, axis=0, keepdims=True)
    si_ref[1:2, :] += jnp.sum(item * item, axis=0, keepdims=True)
    sd_ref[0:1, :] += jnp.sum(dense, axis=0, keepdims=True)
    sd_ref[1:2, :] += jnp.sum(dense * dense, axis=0, keepdims=True)
    ss_ref[0:1, :] += jnp.sum(sp, axis=0, keepdims=True)
    ss_ref[1:2, :] += jnp.sum(sp * sp, axis=0, keepdims=True)


def _attention(ids, seq3, item, dense, sp, wq, ws, wm, b1, w2, b2, w3, b3):
    wspec = lambda shape: pl.BlockSpec(shape, lambda i: (0,) * len(shape))
    return pl.pallas_call(
        _att_body,
        grid=(GRID_A,),
        in_specs=[
            pl.BlockSpec((L, BT), lambda i: (0, i)),
            pl.BlockSpec((BT, L, DSEQ), lambda i: (i, 0, 0)),
            pl.BlockSpec((BT, DSEQ), lambda i: (i, 0)),
            pl.BlockSpec((BT, ND), lambda i: (i, 0)),
            pl.BlockSpec((BT, NS * D), lambda i: (i, 0)),
            wspec((DSEQ, ATT1)), wspec((DSEQ, ATT1)), wspec((DSEQ, ATT1)),
            wspec((1, ATT1)), wspec((ATT1, ATT2)), wspec((1, ATT2)),
            wspec((1, ATT2)), wspec((1, 1)),
        ],
        out_specs=[
            pl.BlockSpec((BT, DSEQ), lambda i: (i, 0)),
            pl.BlockSpec((2, DSEQ), lambda i: (0, 0)),
            pl.BlockSpec((2, DSEQ), lambda i: (0, 0)),
            pl.BlockSpec((2, ND), lambda i: (0, 0)),
            pl.BlockSpec((2, NS * D), lambda i: (0, 0)),
        ],
        out_shape=[
            jax.ShapeDtypeStruct((B, DSEQ), jnp.float32),
            jax.ShapeDtypeStruct((2, DSEQ), jnp.float32),
            jax.ShapeDtypeStruct((2, DSEQ), jnp.float32),
            jax.ShapeDtypeStruct((2, ND), jnp.float32),
            jax.ShapeDtypeStruct((2, NS * D), jnp.float32),
        ],
    )(ids, seq3, item, dense, sp, wq, ws, wm, b1, w2, b2, w3, b3)


BT2 = 512
GRID_B = B // BT2


def _mlp_body(user_ref, item_ref, dense_ref, sp_ref,
              su_ref, si_ref, sd_ref, ss_ref,
              gu_ref, gi_ref, gd_ref, gs_ref,
              bu_ref, bi_ref, bd_ref, bs_ref,
              w1u_ref, w1i_ref, w1d_ref, w1s_ref, b1_ref, a1_ref,
              w2_ref, b2_ref, a2_ref, wf_ref, bf_ref, out_ref):
    inv_b = 1.0 / B

    def norm(x, s_ref, g_ref, b_ref):
        mu = s_ref[0:1, :] * inv_b
        var = s_ref[1:2, :] * inv_b - mu * mu
        return g_ref[...] * (x - mu) * lax.rsqrt(var + 1e-3) + b_ref[...]

    xu = norm(user_ref[...], su_ref, gu_ref, bu_ref)
    xi = norm(item_ref[...], si_ref, gi_ref, bi_ref)
    xd = norm(dense_ref[...], sd_ref, gd_ref, bd_ref)
    xs = norm(sp_ref[...], ss_ref, gs_ref, bs_ref)
    h = (jnp.dot(xu, w1u_ref[...], preferred_element_type=jnp.float32)
         + jnp.dot(xi, w1i_ref[...], preferred_element_type=jnp.float32)
         + jnp.dot(xd, w1d_ref[...], preferred_element_type=jnp.float32)
         + jnp.dot(xs, w1s_ref[...], preferred_element_type=jnp.float32)
         + b1_ref[...])
    h = jnp.maximum(h, 0.0) + a1_ref[...] * jnp.minimum(h, 0.0)
    h = jnp.dot(h, w2_ref[...], preferred_element_type=jnp.float32) + b2_ref[...]
    h = jnp.maximum(h, 0.0) + a2_ref[...] * jnp.minimum(h, 0.0)
    o = jnp.dot(h, wf_ref[...], preferred_element_type=jnp.float32) + bf_ref[0, 0]
    out_ref[...] = 1.0 / (1.0 + jnp.exp(-o))


def _mlp(user, item, dense, sp, su, si, sd, ss, gb, w1p, b1, a1, w2, b2, a2, wf, bf):
    gu, gi, gd, gs, bu, bi, bd, bs = gb
    w1u, w1i, w1d, w1s = w1p
    wspec = lambda shape: pl.BlockSpec(shape, lambda i: (0,) * len(shape))
    return pl.pallas_call(
        _mlp_body,
        grid=(GRID_B,),
        in_specs=[
            pl.BlockSpec((BT2, DSEQ), lambda i: (i, 0)),
            pl.BlockSpec((BT2, DSEQ), lambda i: (i, 0)),
            pl.BlockSpec((BT2, ND), lambda i: (i, 0)),
            pl.BlockSpec((BT2, NS * D), lambda i: (i, 0)),
            wspec((2, DSEQ)), wspec((2, DSEQ)), wspec((2, ND)), wspec((2, NS * D)),
            wspec((1, DSEQ)), wspec((1, DSEQ)), wspec((1, ND)), wspec((1, NS * D)),
            wspec((1, DSEQ)), wspec((1, DSEQ)), wspec((1, ND)), wspec((1, NS * D)),
            wspec((DSEQ, F1)), wspec((DSEQ, F1)), wspec((ND, F1)), wspec((NS * D, F1)),
            wspec((1, F1)), wspec((1, F1)),
            wspec((F1, F2)), wspec((1, F2)), wspec((1, F2)),
            wspec((F2, 1)), wspec((1, 1)),
        ],
        out_specs=pl.BlockSpec((BT2, 1), lambda i: (i, 0)),
        out_shape=jax.ShapeDtypeStruct((B, 1), jnp.float32),
    )(user, item, dense, sp, su, si, sd, ss, gu, gi, gd, gs, bu, bi, bd, bs,
      w1u, w1i, w1d, w1s, b1, a1, w2, b2, a2, wf, bf)


def kernel(dense_inputs, sparse_inputs, seq_inputs, item_inputs, W_beh, W_sparse,
           W_att1, b_att1, W_att2, b_att2, W_att3, b_att3, gamma, beta,
           W_f1, b_f1, a1, W_f2, b_f2, a2, W_final, b_final):
    tbeh = _repack(W_beh.transpose(0, 2, 1), BEH, VB, VPB).reshape(BEH * VPB, D)
    tsp = _repack(W_sparse.transpose(0, 2, 1), NS, VS, VPS).reshape(NS * VPS, D)
    beh_off = (jnp.arange(BEH, dtype=jnp.int32) * VPB)
    bcol = jnp.arange(B, dtype=jnp.int32)[None, :]
    NBT = B // 128
    seq4 = seq_inputs.astype(jnp.int32).reshape(NBT, 128, L, BEH).transpose(2, 0, 3, 1)
    idx_seq = (seq4 + beh_off[None, None, :, None]).reshape(-1)
    l_ = jnp.arange(L, dtype=jnp.int32)[:, None, None, None]
    bt_ = jnp.arange(NBT, dtype=jnp.int32)[None, :, None, None]
    beh_ = jnp.arange(BEH, dtype=jnp.int32)[None, None, :, None]
    bl_ = jnp.arange(128, dtype=jnp.int32)[None, None, None, :]
    oidx_seq = ((bt_ * 128 + bl_) * (L * BEH) + l_ * BEH + beh_).reshape(-1)
    idx_it = (item_inputs.astype(jnp.int32).T + beh_off[:, None]).reshape(-1)
    oidx_it = (bcol * BEH + jnp.arange(BEH, dtype=jnp.int32)[:, None]).reshape(-1)
    sp_off = (jnp.arange(NS, dtype=jnp.int32) * VPS)
    idx_sp = (sparse_inputs.astype(jnp.int32).T + sp_off[:, None]).reshape(-1)
    oidx_sp = (bcol * NS + jnp.arange(NS, dtype=jnp.int32)[:, None]).reshape(-1)

    rows_seq, rows_sp, rows_it = _sc_gather(
        tbeh, tsp, idx_seq, oidx_seq, idx_sp, oidx_sp, idx_it, oidx_it)
    seq3 = rows_seq.reshape(B, L, DSEQ)
    item = rows_it.reshape(B, DSEQ)
    sp = rows_sp.reshape(B, NS * D)

    W1 = W_att1
    wq = W1[0:DSEQ] + W1[2 * DSEQ:3 * DSEQ]
    ws = W1[DSEQ:2 * DSEQ] - W1[2 * DSEQ:3 * DSEQ]
    wm = W1[3 * DSEQ:4 * DSEQ]
    ids = seq_inputs[:, :, 0].astype(jnp.int32).T

    user, su, si, sd, ss = _attention(
        ids, seq3, item, dense_inputs, sp,
        wq, ws, wm, b_att1.reshape(1, ATT1), W_att2, b_att2.reshape(1, ATT2),
        W_att3.reshape(1, ATT2), b_att3.reshape(1, 1))

    o0, o1, o2, o3 = 0, DSEQ, 2 * DSEQ, 2 * DSEQ + ND
    gb = (gamma[o0:o1].reshape(1, -1), gamma[o1:o2].reshape(1, -1),
          gamma[o2:o3].reshape(1, -1), gamma[o3:].reshape(1, -1),
          beta[o0:o1].reshape(1, -1), beta[o1:o2].reshape(1, -1),
          beta[o2:o3].reshape(1, -1), beta[o3:].reshape(1, -1))
    w1p = (W_f1[o0:o1], W_f1[o1:o2], W_f1[o2:o3], W_f1[o3:])

    return _mlp(user, item, dense_inputs, sp, su, si, sd, ss, gb, w1p,
                b_f1.reshape(1, F1), a1.reshape(1, F1), W_f2,
                b_f2.reshape(1, F2), a2.reshape(1, F2), W_final,
                b_final.reshape(1, 1))
